# baseline (speedup 1.0000x reference)
import functools

import jax
import jax.numpy as jnp
from jax import lax
from jax.experimental import pallas as pl
from jax.experimental.pallas import tpu as pltpu
from jax.experimental.pallas import tpu_sc as plsc

N = 100000
E = 100000
F = 128
H = 16
D = 16

NC, NS = 2, 16
NW = NC * NS
GCH = 128
GPW = 25
EP = NW * GPW * GCH
SCH = 50
VGRP = 10
HALF = N // 2
RPT2 = 3136
HROWS = NS * RPT2
ZCH = 448

HI = lax.Precision.HIGHEST
BN = 1000
BEDGE = 2048


def _silu(x):
    return x * jax.nn.sigmoid(x)


def _prep_body(h_ref, cp_ref, wa_ref, wb_ref, tr_ref, tc_ref):
    hb = h_ref[...]
    cp = cp_ref[...]
    z12 = jnp.zeros((hb.shape[0], 12), jnp.float32)
    tr_ref[...] = jnp.concatenate(
        [jnp.dot(hb, wa_ref[...], precision=HI), cp, z12], axis=1)
    tc_ref[...] = jnp.concatenate(
        [jnp.dot(hb, wb_ref[...], precision=HI), cp, z12], axis=1)


_prep_call = pl.pallas_call(
    _prep_body,
    grid=(N // BN,),
    in_specs=[
        pl.BlockSpec((BN, F), lambda i: (i, 0)),
        pl.BlockSpec((BN, 4), lambda i: (i, 0)),
        pl.BlockSpec((F, H), lambda i: (0, 0)),
        pl.BlockSpec((F, H), lambda i: (0, 0)),
    ],
    out_specs=[pl.BlockSpec((BN, 32), lambda i: (i, 0))] * 2,
    out_shape=[jax.ShapeDtypeStruct((N, 32), jnp.float32)] * 2,
)


@functools.cache
def _gather_kernel():
    mesh = plsc.VectorSubcoreMesh(core_axis_name="c", subcore_axis_name="s")

    @functools.partial(
        pl.kernel,
        mesh=mesh,
        out_type=[jax.ShapeDtypeStruct((EP, 32), jnp.float32)] * 2,
        scratch_types=[
            pltpu.VMEM((GPW, GCH), jnp.int32),
            pltpu.VMEM((GPW, GCH), jnp.int32),
            pltpu.VMEM((GCH, 32), jnp.float32),
            pltpu.VMEM((GCH, 32), jnp.float32),
            pltpu.SemaphoreType.DMA,
            pltpu.SemaphoreType.DMA,
        ],
        compiler_params=pltpu.CompilerParams(use_tc_tiling_on_sc=False),
    )
    def gather(tr_hbm, tc_hbm, ridx_hbm, cidx_hbm, gr_hbm, gc_hbm,
               ridx_v, cidx_v, bufr, bufc, semr, semc):
        cid = lax.axis_index("c")
        sid = lax.axis_index("s")
        wid = sid * NC + cid
        pltpu.sync_copy(ridx_hbm.at[wid], ridx_v)
        pltpu.sync_copy(cidx_hbm.at[wid], cidx_v)
        base = wid * (GPW * GCH)

        def body(j, carry):
            cr = pltpu.async_copy(tr_hbm.at[ridx_v.at[j]], bufr, semr)
            cc = pltpu.async_copy(tc_hbm.at[cidx_v.at[j]], bufc, semc)
            cr.wait()
            cc.wait()
            pltpu.sync_copy(bufr, gr_hbm.at[pl.ds(base + j * GCH, GCH)])
            pltpu.sync_copy(bufc, gc_hbm.at[pl.ds(base + j * GCH, GCH)])
            return carry

        lax.fori_loop(0, GPW, body, 0)

    return gather


def _edge_body(gr_ref, gc_ref, ea_ref, w1d_ref, be1_ref, wr_ref,
               we2_ref, be2_ref, ws1_ref, bs1_ref, ws2_ref, bs2_ref,
               wc1_ref, bc1_ref, wc2_ref,
               ef_ref, sc_ref, rel_ref):
    gr = gr_ref[...]
    gc = gc_ref[...]
    X = gr + gc
    Y = gr - gc
    upv = X[:, :16]
    cd4 = Y[:, 16:20]
    radial = jnp.sum(cd4 * cd4, axis=1, keepdims=True)
    pre1 = (upv + radial * wr_ref[...]
            + jnp.dot(ea_ref[...], w1d_ref[...], precision=HI) + be1_ref[...])
    m = _silu(pre1)
    m = _silu(jnp.dot(m, we2_ref[...], precision=HI) + be2_ref[...])
    tp = jnp.concatenate([m * m[:, i:i + 1] for i in range(16)], axis=1)
    s = jax.nn.relu(jnp.dot(tp, ws1_ref[...], precision=HI) + bs1_ref[...])
    ef = jnp.dot(s, ws2_ref[...], precision=HI) + bs2_ref[...]
    t = _silu(jnp.dot(ef, wc1_ref[...], precision=HI) + bc1_ref[...])
    cw = jnp.sum(t * wc2_ref[...], axis=1, keepdims=True)
    tr4 = cd4 * cw
    nrows = gr.shape[0]
    ef_ref[...] = ef
    sc_ref[...] = jnp.concatenate(
        [tr4[:, :3], jnp.ones((nrows, 1), jnp.float32),
         jnp.zeros((nrows, 4), jnp.float32)], axis=1)
    rel_ref[...] = cd4 * (1.0 / (jnp.sqrt(radial) + 1e-8))


def _w_spec(r, c):
    return pl.BlockSpec((r, c), lambda i: (0, 0))


_edge_call = pl.pallas_call(
    _edge_body,
    grid=(EP // BEDGE,),
    in_specs=[
        pl.BlockSpec((BEDGE, 32), lambda i: (i, 0)),
        pl.BlockSpec((BEDGE, 32), lambda i: (i, 0)),
        pl.BlockSpec((BEDGE, D), lambda i: (i, 0)),
        _w_spec(D, H), _w_spec(1, H), _w_spec(1, H),
        _w_spec(H, H), _w_spec(1, H),
        _w_spec(H * H, 2 * H), _w_spec(1, 2 * H),
        _w_spec(2 * H, H), _w_spec(1, H),
        _w_spec(H, H), _w_spec(1, H), _w_spec(1, H),
    ],
    out_specs=[
        pl.BlockSpec((BEDGE, 16), lambda i: (i, 0)),
        pl.BlockSpec((BEDGE, 8), lambda i: (i, 0)),
        pl.BlockSpec((BEDGE, 4), lambda i: (i, 0)),
    ],
    out_shape=[
        jax.ShapeDtypeStruct((EP, 16), jnp.float32),
        jax.ShapeDtypeStruct((EP, 8), jnp.float32),
        jax.ShapeDtypeStruct((EP, 4), jnp.float32),
    ],
)


@functools.cache
def _scatter_kernel():
    mesh = plsc.VectorSubcoreMesh(core_axis_name="c", subcore_axis_name="s")

    @functools.partial(
        pl.kernel,
        mesh=mesh,
        out_type=[
            jax.ShapeDtypeStruct((NC, HROWS, 16), jnp.float32),
            jax.ShapeDtypeStruct((NC, HROWS, 8), jnp.float32),
        ],
        scratch_types=[
            pltpu.VMEM((SCH, GCH), jnp.int32),
            pltpu.VMEM((VGRP * GCH, 16), jnp.float32),
            pltpu.VMEM((VGRP * GCH, 8), jnp.float32),
            pltpu.VMEM_SHARED((HROWS, 16), jnp.float32),
            pltpu.VMEM_SHARED((HROWS, 8), jnp.float32),
        ],
        compiler_params=pltpu.CompilerParams(use_tc_tiling_on_sc=False),
    )
    def scatter(idx_hbm, ef_hbm, sc_hbm, z16_hbm, z8_hbm, agge_hbm, aggc_hbm,
                idx_v, vals16, vals8, acca, accb):
        cid = lax.axis_index("c")
        sid = lax.axis_index("s")
        rbase = sid * RPT2
        for t in range(RPT2 // ZCH):
            pltpu.sync_copy(z16_hbm, acca.at[pl.ds(rbase + t * ZCH, ZCH)])
            pltpu.sync_copy(z8_hbm, accb.at[pl.ds(rbase + t * ZCH, ZCH)])
        plsc.subcore_barrier()
        pltpu.sync_copy(idx_hbm.at[cid, sid], idx_v)
        ebase = sid * (SCH * GCH)

        def group(g, carry):
            off = ebase + g * (VGRP * GCH)
            pltpu.sync_copy(ef_hbm.at[pl.ds(off, VGRP * GCH)], vals16)
            pltpu.sync_copy(sc_hbm.at[pl.ds(off, VGRP * GCH)], vals8)
            for j in range(VGRP):
                pltpu.sync_copy(vals16.at[pl.ds(j * GCH, GCH)],
                                acca.at[idx_v.at[g * VGRP + j]], add=True)
                pltpu.sync_copy(vals8.at[pl.ds(j * GCH, GCH)],
                                accb.at[idx_v.at[g * VGRP + j]], add=True)
            return carry

        lax.fori_loop(0, SCH // VGRP, group, 0)
        plsc.subcore_barrier()
        for t in range(RPT2 // ZCH):
            sl = pl.ds(rbase + t * ZCH, ZCH)
            pltpu.sync_copy(acca.at[sl], agge_hbm.at[cid, sl])
            pltpu.sync_copy(accb.at[sl], aggc_hbm.at[cid, sl])

    return scatter


def _node_body(h_ref, cp_ref, rel_ref, agge_ref, aggc_ref,
               wn1h_ref, wn1r_ref, wn1a_ref, bn1_ref,
               wn2a_ref, wn2b_ref, bn2a_ref, bn2b_ref,
               houta_ref, houtb_ref, co_ref):
    hb = h_ref[...]
    rel = rel_ref[...]
    agge = agge_ref[0]
    aggc = aggc_ref[0]
    cnt = jnp.clip(aggc[:, 3:4], 1.0, None)
    co_ref[...] = cp_ref[...] + aggc[:, :4] / cnt
    z = _silu(jnp.dot(hb, wn1h_ref[...], precision=HI)
              + jnp.dot(rel, wn1r_ref[...], precision=HI)
              + jnp.dot(agge, wn1a_ref[...], precision=HI) + bn1_ref[...])
    houta_ref[...] = hb + jnp.dot(z, wn2a_ref[...], precision=HI) + bn2a_ref[...]
    houtb_ref[...] = rel + jnp.dot(z, wn2b_ref[...], precision=HI) + bn2b_ref[...]


_node_call = pl.pallas_call(
    _node_body,
    grid=(N // BN,),
    in_specs=[
        pl.BlockSpec((BN, F), lambda i: (i, 0)),
        pl.BlockSpec((BN, 4), lambda i: (i, 0)),
        pl.BlockSpec((BN, 4), lambda i: (i, 0)),
        pl.BlockSpec((1, BN, H), lambda i: (i // (HALF // BN),
                                            i % (HALF // BN), 0)),
        pl.BlockSpec((1, BN, 8), lambda i: (i // (HALF // BN),
                                            i % (HALF // BN), 0)),
        _w_spec(F, H), _w_spec(4, H), _w_spec(H, H), _w_spec(1, H),
        _w_spec(H, F), _w_spec(H, 4), _w_spec(1, F), _w_spec(1, 4),
    ],
    out_specs=[
        pl.BlockSpec((BN, F), lambda i: (i, 0)),
        pl.BlockSpec((BN, 4), lambda i: (i, 0)),
        pl.BlockSpec((BN, 4), lambda i: (i, 0)),
    ],
    out_shape=[
        jax.ShapeDtypeStruct((N, F), jnp.float32),
        jax.ShapeDtypeStruct((N, 4), jnp.float32),
        jax.ShapeDtypeStruct((N, 4), jnp.float32),
    ],
)


def kernel(h, coord, edge_attr, W_e1, b_e1, W_e2, b_e2, W_s1, b_s1, W_s2,
           b_s2, W_n1, b_n1, W_n2, b_n2, W_c1, b_c1, W_c2, edge_index):
    f32 = jnp.float32
    row = edge_index[0].astype(jnp.int32)
    col = edge_index[1].astype(jnp.int32)
    pad = EP - E
    ridx = jnp.concatenate([row, jnp.zeros((pad,), jnp.int32)]).reshape(
        NW, GPW, GCH)
    cidx = jnp.concatenate([col, jnp.zeros((pad,), jnp.int32)]).reshape(
        NW, GPW, GCH)
    rowp = jnp.concatenate([row, jnp.full((pad,), N, jnp.int32)])
    sidx = jnp.stack([
        jnp.where((rowp >= c * HALF) & (rowp < (c + 1) * HALF),
                  rowp - c * HALF, HALF).reshape(NS, SCH, GCH)
        for c in range(NC)])
    z16 = jnp.zeros((ZCH, 16), f32)
    z8 = jnp.zeros((ZCH, 8), f32)
    coordp = jnp.pad(coord.astype(f32), ((0, 0), (0, 1)))
    eap = jnp.concatenate(
        [edge_attr.astype(f32), jnp.zeros((pad, D), f32)], axis=0)

    tr_tab, tc_tab = _prep_call(h, coordp, W_e1[:F], W_e1[F:2 * F])
    grow, gcol = _gather_kernel()(tr_tab, tc_tab, ridx, cidx)
    ef, sc16, rel4 = _edge_call(
        grow, gcol, eap,
        W_e1[2 * F + 1:], b_e1.reshape(1, H), W_e1[2 * F].reshape(1, H),
        W_e2, b_e2.reshape(1, H),
        W_s1, b_s1.reshape(1, 2 * H),
        W_s2, b_s2.reshape(1, H),
        W_c1, b_c1.reshape(1, H), W_c2.reshape(1, H),
    )
    agge, aggc = _scatter_kernel()(sidx, ef, sc16, z16, z8)

    wn1r = jnp.pad(W_n1[F:F + 3], ((0, 1), (0, 0)))
    wn2b = jnp.pad(W_n2[:, F:], ((0, 0), (0, 1)))
    bn2b = jnp.pad(b_n2[F:], (0, 1))
    houta, houtb, co4 = _node_call(
        h, coordp, rel4, agge, aggc,
        W_n1[:F], wn1r, W_n1[F + 3:], b_n1.reshape(1, H),
        W_n2[:, :F], wn2b, b_n2[:F].reshape(1, F), bn2b.reshape(1, 4),
    )
    h_out = jnp.concatenate([houta, houtb[:, :3]], axis=1)
    return h_out, co4[:, :3], edge_attr
